# baseline (speedup 1.0000x reference)
import functools

import jax
import jax.numpy as jnp
from jax.experimental import pallas as pl
from jax.experimental.pallas import tpu as pltpu
from jax.experimental.pallas import tpu_sc as plsc

E = 8
TOP_K = 2
D_MODEL = 768
D_FF = 3072
T = 2048
NA = T * TOP_K
TILE = 256
NT = (NA + E * (TILE - 1) + TILE - 1) // TILE
P = NT * TILE

SC_SUBCORES = 16
SC_CHUNK = 128
SENTINEL = T << 16


def _router_kernel(x_ref, wr_ref, pos_ref, val_ref, counts_ref):
    x = x_ref[...]
    logits = jnp.dot(x, wr_ref[...], preferred_element_type=jnp.float32)
    m = jnp.max(logits, axis=-1, keepdims=True)
    ex = jnp.exp(logits - m)
    probs = ex / jnp.sum(ex, axis=-1, keepdims=True)

    iota = jax.lax.broadcasted_iota(jnp.int32, probs.shape, 1)
    p1 = jnp.max(probs, axis=-1, keepdims=True)
    i1 = jnp.min(jnp.where(probs == p1, iota, E), axis=-1, keepdims=True)
    masked = jnp.where(iota == i1, -jnp.inf, probs)
    p2 = jnp.max(masked, axis=-1, keepdims=True)
    i2 = jnp.min(jnp.where(masked == p2, iota, E), axis=-1, keepdims=True)
    o1 = (iota == i1).astype(jnp.float32)
    o2 = (iota == i2).astype(jnp.float32)

    tok = jax.lax.broadcasted_iota(jnp.int32, (T, 1), 0)
    v1 = (tok << 16) | jax.lax.shift_right_logical(
        jax.lax.bitcast_convert_type(p1, jnp.int32), 16)
    v2 = (tok << 16) | jax.lax.shift_right_logical(
        jax.lax.bitcast_convert_type(p2, jnp.int32), 16)
    val_ref[...] = jnp.concatenate([v1, v2], axis=0)

    counts = jnp.sum(o1, axis=0, keepdims=True) + jnp.sum(o2, axis=0, keepdims=True)
    counts_ref[...] = counts

    padded = jnp.floor((counts + (TILE - 1)) * (1.0 / TILE)) * TILE
    pincl = padded
    s = 1
    while s < E:
        shifted = jnp.concatenate(
            [jnp.zeros((1, s), jnp.float32),
             jax.lax.slice(pincl, (0, 0), (1, E - s))], axis=1)
        pincl = pincl + shifted
        s *= 2
    poffset = pincl - padded

    rb = jax.lax.broadcasted_iota(jnp.int32, (TILE, TILE), 0)
    cb = jax.lax.broadcasted_iota(jnp.int32, (TILE, TILE), 1)
    tri = (rb > cb).astype(jnp.float32)
    carry = jnp.zeros((1, E), dtype=jnp.float32)
    nblk = T // TILE
    for half, o in enumerate((o1, o2)):
        for b in range(nblk):
            sl = jax.lax.slice(o, (b * TILE, 0), ((b + 1) * TILE, E))
            excl = jnp.dot(tri, sl, preferred_element_type=jnp.float32) + carry
            rank = jnp.sum(excl * sl, axis=1, keepdims=True)
            offs = jnp.sum(poffset * sl, axis=1, keepdims=True)
            pos_ref[pl.ds(half * T + b * TILE, TILE), :] = (
                rank + offs).astype(jnp.int32)
            carry = carry + jnp.sum(sl, axis=0, keepdims=True)


def _perm_sc_kernel(idx_hbm, val_hbm, init_hbm, out_hbm, idx_v, val_v):
    sid = jax.lax.axis_index("s")
    ini = P // SC_SUBCORES
    pltpu.sync_copy(init_hbm.at[pl.ds(sid * ini, ini)],
                    out_hbm.at[pl.ds(sid * ini, ini)])
    plsc.subcore_barrier()
    pltpu.sync_copy(idx_hbm.at[sid], idx_v)
    pltpu.sync_copy(val_hbm.at[sid], val_v)
    for r in range(NA // SC_SUBCORES // SC_CHUNK):
        pltpu.sync_copy(val_v.at[r], out_hbm.at[idx_v.at[r]])


_perm_sc = functools.partial(
    pl.kernel,
    out_type=jax.ShapeDtypeStruct((P,), jnp.int32),
    mesh=plsc.VectorSubcoreMesh(
        core_axis_name="c", subcore_axis_name="s", num_cores=1),
    scratch_types=[
        pltpu.VMEM((NA // SC_SUBCORES // SC_CHUNK, SC_CHUNK), jnp.int32),
        pltpu.VMEM((NA // SC_SUBCORES // SC_CHUNK, SC_CHUNK), jnp.int32),
    ],
)(_perm_sc_kernel)


def _ffn_kernel(meta_ref, x_ref, perm_ref, win_ref, wout_ref, out_ref):
    i = pl.program_id(0)

    @pl.when(i == 0)
    def _():
        out_ref[...] = jnp.zeros_like(out_ref)

    n_active = meta_ref[NT]

    @pl.when(i < n_active)
    def _():
        pv = perm_ref[0]
        tok_ids = jax.lax.shift_right_logical(pv, 16)
        g = jax.lax.bitcast_convert_type(pv << 16, jnp.float32)
        tok_iota = jax.lax.broadcasted_iota(jnp.int32, (T, TILE), 0)
        mt = (tok_ids == tok_iota).astype(jnp.float32)
        xt = jax.lax.dot_general(mt, x_ref[...], (((0,), (0,)), ((), ())),
                                 preferred_element_type=jnp.float32)
        h = jnp.dot(xt, win_ref[0], preferred_element_type=jnp.float32)
        h = jnp.maximum(h, 0.0)
        y = jnp.dot(h, wout_ref[0], preferred_element_type=jnp.float32)
        out_ref[...] += jnp.dot(mt * g, y, preferred_element_type=jnp.float32)


@jax.jit
def kernel(input_batch, W_router, W_in, W_out):
    b, s, d = input_batch.shape
    x = input_batch.reshape(-1, d)

    pos, val, counts = pl.pallas_call(
        _router_kernel,
        out_shape=[
            jax.ShapeDtypeStruct((NA, 1), jnp.int32),
            jax.ShapeDtypeStruct((NA, 1), jnp.int32),
            jax.ShapeDtypeStruct((1, E), jnp.float32),
        ],
    )(x, W_router)

    nrow = NA // SC_SUBCORES // SC_CHUNK
    sentinel = jnp.full((P,), SENTINEL, dtype=jnp.int32)
    perm = _perm_sc(pos.reshape(SC_SUBCORES, nrow, SC_CHUNK),
                    val.reshape(SC_SUBCORES, nrow, SC_CHUNK),
                    sentinel)
    perm = perm.reshape(NT, 1, TILE)

    c = counts[0].astype(jnp.int32)
    padded = ((c + TILE - 1) // TILE) * TILE
    cum = jnp.cumsum(padded)
    n_active = cum[E - 1] // TILE
    starts = jnp.arange(NT, dtype=jnp.int32) * TILE
    e_tile = jnp.sum((cum[None, :] <= starts[:, None]).astype(jnp.int32), axis=1)
    last_e = jnp.take(e_tile, n_active - 1)
    e_tile = jnp.where(jnp.arange(NT) < n_active, e_tile, last_e)
    meta = jnp.concatenate([e_tile, n_active[None]]).astype(jnp.int32)

    out = pl.pallas_call(
        _ffn_kernel,
        grid_spec=pltpu.PrefetchScalarGridSpec(
            num_scalar_prefetch=1,
            grid=(NT,),
            in_specs=[
                pl.BlockSpec((T, D_MODEL), lambda i, m: (0, 0)),
                pl.BlockSpec((1, 1, TILE), lambda i, m: (i, 0, 0)),
                pl.BlockSpec((1, D_MODEL, D_FF), lambda i, m: (m[i], 0, 0)),
                pl.BlockSpec((1, D_FF, D_MODEL), lambda i, m: (m[i], 0, 0)),
            ],
            out_specs=pl.BlockSpec((T, D_MODEL), lambda i, m: (0, 0)),
        ),
        out_shape=jax.ShapeDtypeStruct((T, D_MODEL), jnp.float32),
    )(meta, x, perm, W_in, W_out)

    return out.reshape(b, s, d)
